# baseline (speedup 1.0000x reference)
import functools

import jax
import jax.numpy as jnp
from jax import lax
from jax.experimental import pallas as pl
from jax.experimental.pallas import tpu as pltpu
from jax.experimental.pallas import tpu_sc as plsc

EMBED = 64
OUT_D = 65
LANES = 16
NC, NS = 2, 16
NW = NC * NS


def _emb_body(bpw, pitch_hbm, idx_hbm, table_hbm, out_hbm,
              idx_v, pitch_v, rows_v, out_v, sem):
    wid = lax.axis_index("s") * NC + lax.axis_index("c")
    base = wid * bpw
    pltpu.sync_copy(idx_hbm.at[pl.ds(base, bpw)], idx_v)
    pltpu.sync_copy(pitch_hbm.at[pl.ds(base, bpw)], pitch_v)
    pltpu.async_copy(table_hbm.at[idx_v], rows_v, sem).wait()

    def row_body(r, carry):
        for c in range(EMBED // LANES):
            out_v[pl.ds(r * OUT_D + 1 + c * LANES, LANES)] = (
                rows_v[r, pl.ds(c * LANES, LANES)]
            )
        return carry

    lax.fori_loop(0, bpw, row_body, 0)

    def pitch_body(g, carry):
        vals = pitch_v[pl.ds(g * LANES, LANES)]
        fidx = (lax.iota(jnp.int32, LANES) + g * LANES) * OUT_D
        plsc.store_scatter(out_v, [fidx], vals)
        return carry

    lax.fori_loop(0, bpw // LANES, pitch_body, 0)

    pltpu.sync_copy(out_v, out_hbm.at[pl.ds(base * OUT_D, bpw * OUT_D)])


def kernel(pitch, timbre_id, table):
    batch = pitch.shape[0]
    bpw = batch // NW

    mesh = plsc.VectorSubcoreMesh(
        core_axis_name="c", subcore_axis_name="s", num_cores=NC, num_subcores=NS
    )
    run = functools.partial(
        pl.kernel,
        out_type=jax.ShapeDtypeStruct((batch * OUT_D,), jnp.float32),
        mesh=mesh,
        compiler_params=pltpu.CompilerParams(
            needs_layout_passes=False, use_tc_tiling_on_sc=False
        ),
        scratch_types=[
            pltpu.VMEM((bpw,), jnp.int32),
            pltpu.VMEM((bpw,), jnp.float32),
            pltpu.VMEM((bpw, EMBED), jnp.float32),
            pltpu.VMEM((bpw * OUT_D,), jnp.float32),
            pltpu.SemaphoreType.DMA,
        ],
    )(functools.partial(_emb_body, bpw))
    flat = run(pitch, timbre_id, table)
    return flat.reshape(batch, OUT_D)


# speedup vs baseline: 1.0610x; 1.0610x over previous
import functools

import jax
import jax.numpy as jnp
from jax import lax
from jax.experimental import pallas as pl
from jax.experimental.pallas import tpu as pltpu
from jax.experimental.pallas import tpu_sc as plsc

EMBED = 64
PADW = 128
OUT_D = 65
LANES = 16
NC, NS = 2, 16
NW = NC * NS


def _emb_body(bpw, pitch_hbm, idx_hbm, table_hbm, out_hbm,
              idx_v, pitch_v, rows_v, out_v, sem):
    wid = lax.axis_index("s") * NC + lax.axis_index("c")
    base = wid * bpw
    pltpu.sync_copy(idx_hbm.at[pl.ds(base, bpw)], idx_v)
    pltpu.sync_copy(pitch_hbm.at[pl.ds(base, bpw)], pitch_v)
    pltpu.async_copy(table_hbm.at[idx_v], rows_v, sem).wait()

    def row_body(r, carry):
        for c in range(EMBED // LANES):
            out_v[pl.ds(r * OUT_D + 1 + c * LANES, LANES)] = (
                rows_v[r, pl.ds(c * LANES, LANES)]
            )
        return carry

    lax.fori_loop(0, bpw, row_body, 0)

    def pitch_body(g, carry):
        vals = pitch_v[pl.ds(g * LANES, LANES)]
        fidx = (lax.iota(jnp.int32, LANES) + g * LANES) * OUT_D
        plsc.store_scatter(out_v, [fidx], vals)
        return carry

    lax.fori_loop(0, bpw // LANES, pitch_body, 0)

    pltpu.sync_copy(out_v, out_hbm.at[pl.ds(base * OUT_D, bpw * OUT_D)])


def kernel(pitch, timbre_id, table):
    batch = pitch.shape[0]
    bpw = batch // NW

    table_p = jnp.pad(table, ((0, 0), (0, PADW - EMBED)))

    mesh = plsc.VectorSubcoreMesh(
        core_axis_name="c", subcore_axis_name="s", num_cores=NC, num_subcores=NS
    )
    run = functools.partial(
        pl.kernel,
        out_type=jax.ShapeDtypeStruct((batch * OUT_D,), jnp.float32),
        mesh=mesh,
        compiler_params=pltpu.CompilerParams(
            needs_layout_passes=False, use_tc_tiling_on_sc=True
        ),
        scratch_types=[
            pltpu.VMEM((bpw,), jnp.int32),
            pltpu.VMEM((bpw,), jnp.float32),
            pltpu.VMEM((bpw, PADW), jnp.float32),
            pltpu.VMEM((bpw * OUT_D,), jnp.float32),
            pltpu.SemaphoreType.DMA,
        ],
    )(functools.partial(_emb_body, bpw))
    flat = run(pitch, timbre_id, table_p)
    return flat.reshape(batch, OUT_D)


# speedup vs baseline: 1.1440x; 1.0782x over previous
import functools

import jax
import jax.numpy as jnp
from jax import lax
from jax.experimental import pallas as pl
from jax.experimental.pallas import tpu as pltpu
from jax.experimental.pallas import tpu_sc as plsc

EMBED = 64
PADW = 128
OUT_D = 65
LANES = 16
NC, NS = 2, 16
NW = NC * NS
CH = 256


def _emb_body(bpw, pitch_hbm, idx_hbm, table_hbm, out_hbm,
              idx_v, pitch_v, rows_v, out_v, sem):
    wid = lax.axis_index("s") * NC + lax.axis_index("c")
    base = wid * bpw
    iota = lax.iota(jnp.int32, LANES)

    def chunk_body(k, carry):
        cbase = base + k * CH
        pltpu.sync_copy(idx_hbm.at[pl.ds(cbase, CH)], idx_v)
        pltpu.sync_copy(pitch_hbm.at[pl.ds(cbase, CH)], pitch_v)
        pltpu.async_copy(table_hbm.at[idx_v], rows_v, sem).wait()

        def row_body(r, c2):
            rsplat = jnp.full((LANES,), 0, jnp.int32) + r
            for c in range(EMBED // LANES):
                vals = rows_v[r, pl.ds(c * LANES, LANES)]
                jvec = iota + (1 + c * LANES)
                plsc.store_scatter(out_v, [jvec, rsplat], vals)
            return c2

        lax.fori_loop(0, CH, row_body, 0)

        def pitch_body(g, c2):
            out_v[0, pl.ds(g * LANES, LANES)] = pitch_v[pl.ds(g * LANES, LANES)]
            return c2

        lax.fori_loop(0, CH // LANES, pitch_body, 0)

        pltpu.sync_copy(out_v, out_hbm.at[:, pl.ds(cbase, CH)])
        return carry

    lax.fori_loop(0, bpw // CH, chunk_body, 0)


def kernel(pitch, timbre_id, table):
    batch = pitch.shape[0]
    bpw = batch // NW

    table_p = jnp.pad(table, ((0, 0), (0, PADW - EMBED)))

    mesh = plsc.VectorSubcoreMesh(
        core_axis_name="c", subcore_axis_name="s", num_cores=NC, num_subcores=NS
    )
    run = functools.partial(
        pl.kernel,
        out_type=jax.ShapeDtypeStruct((OUT_D, batch), jnp.float32),
        mesh=mesh,
        compiler_params=pltpu.CompilerParams(
            needs_layout_passes=False, use_tc_tiling_on_sc=True
        ),
        scratch_types=[
            pltpu.VMEM((CH,), jnp.int32),
            pltpu.VMEM((CH,), jnp.float32),
            pltpu.VMEM((CH, PADW), jnp.float32),
            pltpu.VMEM((OUT_D, CH), jnp.float32),
            pltpu.SemaphoreType.DMA,
        ],
    )(functools.partial(_emb_body, bpw))
    out_t = run(pitch, timbre_id, table_p)
    return out_t.T


# speedup vs baseline: 1.2552x; 1.0972x over previous
import functools

import jax
import jax.numpy as jnp
from jax import lax
from jax.experimental import pallas as pl
from jax.experimental.pallas import tpu as pltpu
from jax.experimental.pallas import tpu_sc as plsc

EMBED = 64
PADW = 128
OUT_D = 65
LANES = 16
NC, NS = 2, 16
NW = NC * NS
CH = 256
UNROLL = 4


def _emb_body(bpw, pitch_hbm, idx_hbm, table_hbm, out_hbm,
              idx_v, pitch_v, rows_v, out_v, sem):
    wid = lax.axis_index("s") * NC + lax.axis_index("c")
    base = wid * bpw
    iota = lax.iota(jnp.int32, LANES)

    def chunk_body(k, carry):
        cbase = base + k * CH
        pltpu.sync_copy(idx_hbm.at[pl.ds(cbase, CH)], idx_v)
        pltpu.sync_copy(pitch_hbm.at[pl.ds(cbase, CH)], pitch_v)
        pltpu.async_copy(table_hbm.at[idx_v], rows_v, sem).wait()

        def row_body(g, c2):
            for u in range(UNROLL):
                r = g * UNROLL + u
                for c in range(EMBED // LANES):
                    out_v[r, pl.ds(1 + c * LANES, LANES)] = (
                        rows_v[r, pl.ds(c * LANES, LANES)]
                    )
            return c2

        lax.fori_loop(0, CH // UNROLL, row_body, 0)

        zeros = jnp.zeros((LANES,), jnp.int32)

        def pitch_body(g, c2):
            vals = pitch_v[pl.ds(g * LANES, LANES)]
            ridx = iota + g * LANES
            plsc.store_scatter(out_v, [ridx, zeros], vals)
            return c2

        lax.fori_loop(0, CH // LANES, pitch_body, 0)

        pltpu.sync_copy(out_v, out_hbm.at[pl.ds(cbase, CH)])
        return carry

    lax.fori_loop(0, bpw // CH, chunk_body, 0)


def kernel(pitch, timbre_id, table):
    batch = pitch.shape[0]
    bpw = batch // NW

    table_p = jnp.pad(table, ((0, 0), (0, PADW - EMBED)))

    mesh = plsc.VectorSubcoreMesh(
        core_axis_name="c", subcore_axis_name="s", num_cores=NC, num_subcores=NS
    )
    run = functools.partial(
        pl.kernel,
        out_type=jax.ShapeDtypeStruct((batch, OUT_D), jnp.float32),
        mesh=mesh,
        compiler_params=pltpu.CompilerParams(
            needs_layout_passes=False, use_tc_tiling_on_sc=True
        ),
        scratch_types=[
            pltpu.VMEM((CH,), jnp.int32),
            pltpu.VMEM((CH,), jnp.float32),
            pltpu.VMEM((CH, PADW), jnp.float32),
            pltpu.VMEM((CH, OUT_D), jnp.float32),
            pltpu.SemaphoreType.DMA,
        ],
    )(functools.partial(_emb_body, bpw))
    return run(pitch, timbre_id, table_p)
